# baseline (speedup 1.0000x reference)
import functools

import jax
import jax.numpy as jnp
from jax import lax
from jax.experimental import pallas as pl
from jax.experimental.pallas import tpu as pltpu
from jax.experimental.pallas import tpu_sc as plsc

NC = 2
NS = 16
NW = NC * NS
L = 16
B = 16384
D = 64
BPW = B // NW
CHUNK = 128
NCHUNK = BPW // CHUNK
GROUPS = BPW // L

_mesh = plsc.VectorSubcoreMesh(core_axis_name="c", subcore_axis_name="s")


@functools.partial(
    pl.kernel,
    out_type=jax.ShapeDtypeStruct((B,), jnp.float32),
    mesh=_mesh,
    compiler_params=pltpu.CompilerParams(
        needs_layout_passes=False, use_tc_tiling_on_sc=False
    ),
    scratch_types=[
        pltpu.VMEM((3, NCHUNK, CHUNK), jnp.int32),
        pltpu.VMEM((BPW, D), jnp.float32),
        pltpu.VMEM((BPW, D), jnp.float32),
        pltpu.VMEM((BPW, D), jnp.float32),
        pltpu.VMEM((BPW,), jnp.float32),
        pltpu.SemaphoreType.DMA,
    ],
)
def _sc_dots(x_hbm, user_hbm, item_hbm, out_hbm, idx_v, ru, ri, rj, xout, sem):
    wid = lax.axis_index("s") * NC + lax.axis_index("c")
    base = wid * BPW

    pltpu.sync_copy(x_hbm.at[wid], idx_v)

    ru2, ri2, rj2 = ru, ri, rj
    handles = []
    for c in range(NCHUNK):
        dst = pl.ds(c * CHUNK, CHUNK)
        handles.append(pltpu.async_copy(user_hbm.at[idx_v.at[0, c]], ru2.at[dst], sem))
        handles.append(pltpu.async_copy(item_hbm.at[idx_v.at[1, c]], ri2.at[dst], sem))
        handles.append(pltpu.async_copy(item_hbm.at[idx_v.at[2, c]], rj2.at[dst], sem))
    for h in handles:
        h.wait()

    lane = lax.iota(jnp.int32, L)

    def group_body(g, carry):
        vec = jnp.zeros((L,), jnp.float32)
        for k in range(L):
            r = g * L + k
            p = jnp.zeros((L,), jnp.float32)
            for f in range(0, D, L):
                fs = pl.ds(f, L)
                p = p + ru2[r, fs] * (ri2[r, fs] - rj2[r, fs])
            vec = jnp.where(lane == k, jnp.sum(p), vec)
        xout[pl.ds(g * L, L)] = vec
        return carry

    lax.fori_loop(0, GROUPS, group_body, 0)
    pltpu.sync_copy(xout, out_hbm.at[pl.ds(base, BPW)])


def _loss_body(x_ref, o_ref):
    x = x_ref[...]
    sp = jnp.maximum(-x, 0.0) + jnp.log(1.0 + jnp.exp(-jnp.abs(x)))
    o_ref[...] = jnp.sum(sp, keepdims=True)


def kernel(x, user_embeddings, item_embeddings):
    x = x.astype(jnp.int32).reshape(3, NW, NCHUNK, CHUNK).transpose(1, 0, 2, 3)
    x_uij = _sc_dots(x, user_embeddings, item_embeddings)
    loss = pl.pallas_call(
        _loss_body,
        out_shape=jax.ShapeDtypeStruct((1, 1), jnp.float32),
    )(x_uij.reshape(B // 128, 128))
    return loss[0, 0]
